# baseline (speedup 1.0000x reference)
import functools

import jax
import jax.numpy as jnp
from jax import lax
from jax.experimental import pallas as pl
from jax.experimental.pallas import tpu as pltpu
from jax.experimental.pallas import tpu_sc as plsc

D = 64
V = 1_000_000
SCALE = 8.0

NC = 2
NS = 16
NW = NC * NS

L = 16

U_COLS = 256
U_ROWS = U_COLS // 2
N_UNITS = (V - V % 128) // U_COLS
V_TAIL = V % 128
PACKED_ROWS = (V + 1) // 2

_mesh = plsc.VectorSubcoreMesh(
    core_axis_name="c", subcore_axis_name="s", num_cores=NC, num_subcores=NS)


def _iota16():
    return jnp.arange(L, dtype=jnp.int32)


@functools.partial(
    pl.kernel,
    out_type=jax.ShapeDtypeStruct((PACKED_ROWS, 2 * D), jnp.float32),
    mesh=_mesh,
    compiler_params=pltpu.CompilerParams(use_tc_tiling_on_sc=True, needs_layout_passes=False),
    scratch_types=[
        pltpu.VMEM((2, 8, 8, U_COLS), jnp.float32),
        pltpu.VMEM((2, U_ROWS, 2 * D), jnp.float32),
        pltpu.VMEM((8, 8, V_TAIL), jnp.float32),
        pltpu.SemaphoreType.DMA,
        pltpu.SemaphoreType.DMA,
        pltpu.SemaphoreType.DMA,
        pltpu.SemaphoreType.DMA,
    ],
)
def _pack(tbl_t, tail_t, tbl2, stage, outbuf, tstage, g0, g1, o0, o1):
    gsem = (g0, g1)
    osem = (o0, o1)
    cid = lax.axis_index("c")
    sid = lax.axis_index("s")
    wid = sid * NC + cid
    cnt = jnp.where(wid < N_UNITS % NW, N_UNITS // NW + 1, N_UNITS // NW)

    ii = _iota16()
    drv = ii & 7
    dgv = [2 * c + (ii >> 3) for c in range(4)]

    def unit_of(k):
        return wid + k * NW

    def fire_reads(k, b):
        u = unit_of(k)
        for dg in range(8):
            pltpu.async_copy(
                tbl_t.at[pl.ds(dg * 8, 8), pl.ds(u * U_COLS, U_COLS)],
                stage.at[b, dg], gsem[b])

    def wait_reads(b):
        for dg in range(8):
            pltpu.make_async_copy(
                tbl_t.at[pl.ds(dg * 8, 8), pl.ds(0, U_COLS)],
                stage.at[b, dg], gsem[b]).wait()

    def fire_write(k, b):
        u = unit_of(k)
        pltpu.async_copy(
            outbuf.at[b], tbl2.at[pl.ds(u * U_ROWS, U_ROWS)], osem[b])

    def wait_write(b):
        pltpu.make_async_copy(
            outbuf.at[b], tbl2.at[pl.ds(0, U_ROWS)], osem[b]).wait()

    def transform(b):
        @plsc.parallel_loop(0, U_ROWS, unroll=4)
        def _row(r):
            voff0 = jnp.full((L,), 2 * r, jnp.int32)
            voffs = (voff0, voff0 + 1)
            for cc in range(8):
                cp, h = cc % 4, cc // 4
                vals = plsc.load_gather(stage.at[b], [dgv[cp], drv, voffs[h]])
                outbuf[b, r, pl.ds(cc * L, L)] = vals * SCALE

    fire_reads(0, 0)

    n_pairs = (N_UNITS // NW + 2) // 2

    @pl.loop(0, n_pairs)
    def _pair(g):
        for b in range(2):
            k = 2 * g + b

            @pl.when(k < cnt)
            def _():
                wait_reads(b)

                @pl.when(k + 1 < cnt)
                def _():
                    fire_reads(k + 1, 1 - b)

                @pl.when(k >= 2)
                def _():
                    wait_write(b)

                transform(b)
                fire_write(k, b)

    wait_write(0)
    wait_write(1)

    @pl.when(wid == NW - 1)
    def _():
        for dg in range(8):
            pltpu.sync_copy(tail_t.at[pl.ds(dg * 8, 8)], tstage.at[dg])

        @pl.loop(0, V_TAIL // 2)
        def _row(r):
            for cc in range(8):
                cp, h = cc % 4, cc // 4
                voff = jnp.full((L,), 2 * r + h, jnp.int32)
                vals = plsc.load_gather(tstage, [dgv[cp], drv, voff])
                outbuf[0, r, pl.ds(cc * L, L)] = vals * SCALE

        pltpu.sync_copy(
            outbuf.at[0, pl.ds(0, V_TAIL // 2)],
            tbl2.at[pl.ds(PACKED_ROWS - V_TAIL // 2, V_TAIL // 2)])


@functools.partial(
    pl.kernel,
    out_type=jax.ShapeDtypeStruct((200, D, 4096), jnp.float32),
    mesh=_mesh,
    compiler_params=pltpu.CompilerParams(use_tc_tiling_on_sc=True, needs_layout_passes=False),
    scratch_types=[
        pltpu.VMEM((200, 128), jnp.int32),
        pltpu.VMEM((8, 128), jnp.int32),
        pltpu.VMEM((3, 128, 2 * D), jnp.float32),
        pltpu.VMEM((2, D, 128), jnp.float32),
        pltpu.SemaphoreType.DMA,
        pltpu.SemaphoreType.DMA,
        pltpu.SemaphoreType.DMA,
        pltpu.SemaphoreType.DMA,
        pltpu.SemaphoreType.DMA,
    ],
)
def _gather(x_t, tbl2, out3, idx_v, js_v, buf, otile, g0, g1, g2, o0, o1):
    gsem = (g0, g1, g2)
    osem = (o0, o1)
    cid = lax.axis_index("c")
    sid = lax.axis_index("s")
    wid = sid * NC + cid

    ii = _iota16()
    rows = [16 * c + ii for c in range(8)]

    pltpu.sync_copy(x_t.at[:, pl.ds(wid * 128, 128)], idx_v)

    def stage_unit(s, gb):
        for c in range(8):
            js_v[gb, pl.ds(c * L, L)] = idx_v[s, pl.ds(c * L, L)] >> 1
        pltpu.async_copy(tbl2.at[js_v.at[gb]], buf.at[gb], gsem[gb])

    def wait_unit(s, gb):
        pltpu.make_async_copy(tbl2.at[js_v.at[gb]], buf.at[gb],
                              gsem[gb]).wait()

    def fire_out(s, ob):
        pltpu.async_copy(otile.at[ob],
                         out3.at[s, :, pl.ds(wid * 128, 128)], osem[ob])

    def wait_out(s, ob):
        pltpu.make_async_copy(otile.at[ob],
                              out3.at[s, :, pl.ds(wid * 128, 128)],
                              osem[ob]).wait()

    def transform(s, gb, ob):
        offs = []
        for c in range(8):
            par = idx_v[s, pl.ds(c * L, L)] & 1
            offs.append(par << 6)

        @plsc.parallel_loop(0, D, unroll=4)
        def _d(d):
            dsplat = jnp.full((L,), d, jnp.int32)
            for c in range(8):
                vals = plsc.load_gather(buf.at[gb], [rows[c], offs[c] + dsplat])
                otile[ob, d, pl.ds(c * L, L)] = vals

    for s0 in range(3):
        stage_unit(s0, s0)

    @pl.loop(0, 204 // 6)
    def _six(g):
        for j in range(6):
            s = 6 * g + j
            gb = j % 3
            ob = j % 2

            @pl.when(s < 200)
            def _():
                wait_unit(s, gb)

                @pl.when(s >= 2)
                def _():
                    wait_out(s - 2, ob)

                transform(s, gb, ob)
                fire_out(s, ob)

                @pl.when(s + 3 < 200)
                def _():
                    stage_unit(s + 3, gb)

    wait_out(198, 0)
    wait_out(199, 1)


def kernel(x, lut_weight):
    x_t = x.T.astype(jnp.int32)
    tbl_t = lut_weight.T
    tail_t = lut_weight[V - V_TAIL:, :].T
    tbl2 = _pack(tbl_t, tail_t)
    out3 = _gather(x_t, tbl2)
    return out3.transpose(2, 0, 1)


# speedup vs baseline: 3.3595x; 3.3595x over previous
import functools

import jax
import jax.numpy as jnp
from jax import lax
from jax.experimental import pallas as pl
from jax.experimental.pallas import tpu as pltpu
from jax.experimental.pallas import tpu_sc as plsc

D = 64
V = 1_000_000
SCALE = 8.0

NC = 2
NS = 16
NW = NC * NS

L = 16

U_COLS = 256
U_ROWS = U_COLS // 2
N_UNITS = (V - V % 128) // U_COLS
V_TAIL = V % 128
PACKED_ROWS = (V + 1) // 2

_mesh = plsc.VectorSubcoreMesh(
    core_axis_name="c", subcore_axis_name="s", num_cores=NC, num_subcores=NS)


def _iota16():
    return jnp.arange(L, dtype=jnp.int32)


@functools.partial(
    pl.kernel,
    out_type=jax.ShapeDtypeStruct((PACKED_ROWS, 2 * D), jnp.float32),
    mesh=_mesh,
    compiler_params=pltpu.CompilerParams(use_tc_tiling_on_sc=True, needs_layout_passes=False),
    scratch_types=[
        pltpu.VMEM((2, 8, 8, U_COLS), jnp.float32),
        pltpu.VMEM((2, U_ROWS, 2 * D), jnp.float32),
        pltpu.VMEM((8, 8, V_TAIL), jnp.float32),
        pltpu.SemaphoreType.DMA,
        pltpu.SemaphoreType.DMA,
        pltpu.SemaphoreType.DMA,
        pltpu.SemaphoreType.DMA,
    ],
)
def _pack(tbl_t, tail_t, tbl2, stage, outbuf, tstage, g0, g1, o0, o1):
    gsem = (g0, g1)
    osem = (o0, o1)
    cid = lax.axis_index("c")
    sid = lax.axis_index("s")
    wid = sid * NC + cid
    cnt = jnp.where(wid < N_UNITS % NW, N_UNITS // NW + 1, N_UNITS // NW)

    ii = _iota16()
    voffs = [[2 * (16 * rc + ii) + h for rc in range(8)] for h in range(2)]
    rowvs = [16 * rc + ii for rc in range(8)]

    def unit_of(k):
        return wid + k * NW

    def fire_reads(k, b):
        u = unit_of(k)
        for dg in range(8):
            pltpu.async_copy(
                tbl_t.at[pl.ds(dg * 8, 8), pl.ds(u * U_COLS, U_COLS)],
                stage.at[b, dg], gsem[b])

    def wait_reads(b):
        for dg in range(8):
            pltpu.make_async_copy(
                tbl_t.at[pl.ds(dg * 8, 8), pl.ds(0, U_COLS)],
                stage.at[b, dg], gsem[b]).wait()

    def fire_write(k, b):
        u = unit_of(k)
        pltpu.async_copy(
            outbuf.at[b], tbl2.at[pl.ds(u * U_ROWS, U_ROWS)], osem[b])

    def wait_write(b):
        pltpu.make_async_copy(
            outbuf.at[b], tbl2.at[pl.ds(0, U_ROWS)], osem[b]).wait()

    cbase = [[h * 64 + 16 * rc + ii for rc in range(8)] for h in range(2)]

    def transform(b):
        @plsc.parallel_loop(0, D, unroll=2)
        def _d(d):
            dgs = jnp.full((L,), d >> 3, jnp.int32)
            drs = jnp.full((L,), d & 7, jnp.int32)
            dspl = jnp.full((L,), d, jnp.int32)
            for h in range(2):
                for rc in range(8):
                    vals = plsc.load_gather(
                        stage.at[b], [dgs, drs, voffs[h][rc]])
                    colv = (cbase[h][rc] + dspl) & 127
                    plsc.store_scatter(
                        outbuf.at[b], [rowvs[rc], colv], vals * SCALE)

    fire_reads(0, 0)

    n_pairs = (N_UNITS // NW + 2) // 2

    @pl.loop(0, n_pairs)
    def _pair(g):
        for b in range(2):
            k = 2 * g + b

            @pl.when(k < cnt)
            def _():
                wait_reads(b)

                @pl.when(k + 1 < cnt)
                def _():
                    fire_reads(k + 1, 1 - b)

                @pl.when(k >= 2)
                def _():
                    wait_write(b)

                transform(b)
                fire_write(k, b)

    wait_write(0)
    wait_write(1)

    @pl.when(wid == NW - 1)
    def _():
        for dg in range(8):
            pltpu.sync_copy(tail_t.at[pl.ds(dg * 8, 8)], tstage.at[dg])

        @pl.loop(0, D)
        def _d(d):
            dgs = jnp.full((L,), d >> 3, jnp.int32)
            drs = jnp.full((L,), d & 7, jnp.int32)
            dspl = jnp.full((L,), d, jnp.int32)
            for h in range(2):
                for rc in range(2):
                    vals = plsc.load_gather(
                        tstage, [dgs, drs, voffs[h][rc]])
                    colv = (cbase[h][rc] + dspl) & 127
                    plsc.store_scatter(
                        outbuf.at[0], [rowvs[rc], colv], vals * SCALE)

        pltpu.sync_copy(
            outbuf.at[0, pl.ds(0, V_TAIL // 2)],
            tbl2.at[pl.ds(PACKED_ROWS - V_TAIL // 2, V_TAIL // 2)])


@functools.partial(
    pl.kernel,
    out_type=jax.ShapeDtypeStruct((200, D, 4096), jnp.float32),
    mesh=_mesh,
    compiler_params=pltpu.CompilerParams(use_tc_tiling_on_sc=True, needs_layout_passes=False),
    scratch_types=[
        pltpu.VMEM((200, 128), jnp.int32),
        pltpu.VMEM((8, 128), jnp.int32),
        pltpu.VMEM((3, 128, 2 * D), jnp.float32),
        pltpu.VMEM((2, D, 128), jnp.float32),
        pltpu.SemaphoreType.DMA,
        pltpu.SemaphoreType.DMA,
        pltpu.SemaphoreType.DMA,
        pltpu.SemaphoreType.DMA,
        pltpu.SemaphoreType.DMA,
    ],
)
def _gather(x_t, tbl2, out3, idx_v, js_v, buf, otile, g0, g1, g2, o0, o1):
    gsem = (g0, g1, g2)
    osem = (o0, o1)
    cid = lax.axis_index("c")
    sid = lax.axis_index("s")
    wid = sid * NC + cid

    ii = _iota16()
    rows = [16 * c + ii for c in range(8)]

    pltpu.sync_copy(x_t.at[:, pl.ds(wid * 128, 128)], idx_v)

    def stage_unit(s, gb):
        for c in range(8):
            js_v[gb, pl.ds(c * L, L)] = idx_v[s, pl.ds(c * L, L)] >> 1
        pltpu.async_copy(tbl2.at[js_v.at[gb]], buf.at[gb], gsem[gb])

    def wait_unit(s, gb):
        pltpu.make_async_copy(tbl2.at[js_v.at[gb]], buf.at[gb],
                              gsem[gb]).wait()

    def fire_out(s, ob):
        pltpu.async_copy(otile.at[ob],
                         out3.at[s, :, pl.ds(wid * 128, 128)], osem[ob])

    def wait_out(s, ob):
        pltpu.make_async_copy(otile.at[ob],
                              out3.at[s, :, pl.ds(wid * 128, 128)],
                              osem[ob]).wait()

    def transform(s, gb, ob):
        base = []
        for c in range(8):
            iv = idx_v[s, pl.ds(c * L, L)]
            base.append(((iv & 1) << 6) + (iv >> 1))

        @plsc.parallel_loop(0, D, unroll=2)
        def _d(d):
            dsplat = jnp.full((L,), d, jnp.int32)
            for c in range(8):
                colv = (base[c] + dsplat) & 127
                vals = plsc.load_gather(buf.at[gb], [rows[c], colv])
                otile[ob, d, pl.ds(c * L, L)] = vals

    for s0 in range(3):
        stage_unit(s0, s0)

    @pl.loop(0, 204 // 6)
    def _six(g):
        for j in range(6):
            s = 6 * g + j
            gb = j % 3
            ob = j % 2

            @pl.when(s < 200)
            def _():
                wait_unit(s, gb)

                @pl.when(s >= 2)
                def _():
                    wait_out(s - 2, ob)

                transform(s, gb, ob)
                fire_out(s, ob)

                @pl.when(s + 3 < 200)
                def _():
                    stage_unit(s + 3, gb)

    wait_out(198, 0)
    wait_out(199, 1)


def kernel(x, lut_weight):
    x_t = x.T.astype(jnp.int32)
    tbl_t = lut_weight.T
    tail_t = lut_weight[V - V_TAIL:, :].T
    tbl2 = _pack(tbl_t, tail_t)
    out3 = _gather(x_t, tbl2)
    return out3.transpose(2, 0, 1)
